# baseline (speedup 1.0000x reference)
import functools

import jax
import jax.numpy as jnp
from jax import lax
from jax.experimental import pallas as pl
from jax.experimental.pallas import tpu as pltpu
from jax.experimental.pallas import tpu_sc as plsc

_BB = 128
_HB = 8
_LANES = 16
_OW = 137


def _body(L, H, B, tasks_per_worker, num_cores,
          xt_hbm, emb_hbm, pos_hbm, out_hbm,
          idx_v, pos_v, rows0, rows1, obuf0, obuf1,
          sem_g0, sem_g1, sem_o0, sem_o1):
  wid = lax.axis_index("s") * num_cores + lax.axis_index("c")
  t0 = wid * tasks_per_worker

  rows = (rows0, rows1)
  obufs = (obuf0, obuf1)
  gsems = (sem_g0, sem_g1)
  osems = (sem_o0, sem_o1)
  n_th = H // _HB

  pltpu.sync_copy(xt_hbm.at[pl.ds(t0 * _BB, tasks_per_worker * _BB)], idx_v)
  pltpu.sync_copy(pos_hbm, pos_v)

  jot = [jnp.arange(_LANES, dtype=jnp.int32) + _LANES * j
         for j in range(_BB // _LANES)]

  def gather_into(buf_i, k):
    koff = lax.rem(k, tasks_per_worker)
    return pltpu.async_copy(
        emb_hbm.at[idx_v.at[pl.ds(koff * _BB, _BB)]],
        rows[buf_i], gsems[buf_i])

  def drain_outs(buf_i):
    for _ in range(n_th):
      pltpu.make_async_copy(
          obufs[buf_i].at[pl.ds(0, _HB), pl.ds(0, _BB)], out_hbm.at[0, 0, 0],
          osems[buf_i]).wait()

  def transpose_add(buf_i, l):
    rbuf = rows[buf_i]
    obuf = obufs[buf_i]
    pv = [pos_v[l, pl.ds(h4 * _LANES, _LANES)] for h4 in range(H // _LANES)]

    def four_b(b4, _):
      b0 = b4 * 4
      for i in range(4):
        b = b0 + i
        bs = jnp.full((_LANES,), b, jnp.int32)
        for h4 in range(H // _LANES):
          v = rbuf[b, pl.ds(h4 * _LANES, _LANES)] + pv[h4]
          plsc.store_scatter(obuf, [jot[h4], bs], v)
      return 0

    lax.fori_loop(0, _BB // 4, four_b, 0)

  def write_out(buf_i, l, tb):
    for th in range(n_th):
      pltpu.async_copy(
          obufs[buf_i].at[pl.ds(th * _HB, _HB), pl.ds(0, _BB)],
          out_hbm.at[l, th, tb], osems[buf_i])

  n_tb = B // _BB

  gather_into(0, 0)

  def pair(tp, _):
    for k_par in range(2):
      k = tp * 2 + k_par
      t = t0 + k
      l = t // n_tb
      tb = lax.rem(t, n_tb)
      bi = k_par

      pltpu.make_async_copy(
          emb_hbm.at[idx_v.at[pl.ds(0, _BB)]], rows[bi], gsems[bi]).wait()

      @pl.when(k >= 2)
      def _():
        drain_outs(bi)

      gather_into(1 - bi, k + 1)
      transpose_add(bi, l)
      write_out(bi, l, tb)
    return 0

  lax.fori_loop(0, tasks_per_worker // 2, pair, 0)

  pltpu.make_async_copy(
      emb_hbm.at[idx_v.at[pl.ds(0, _BB)]], rows[0], gsems[0]).wait()
  drain_outs(0)
  drain_outs(1)


def kernel(x, emb_table, pos_table):
  B, L = x.shape
  V, H = emb_table.shape
  info = plsc.get_sparse_core_info()
  nw = info.num_cores * info.num_subcores
  n_tb = B // _BB
  tasks_per_worker = (L * n_tb) // nw

  mesh = plsc.VectorSubcoreMesh(core_axis_name="c", subcore_axis_name="s")
  body = functools.partial(_body, L, H, B, tasks_per_worker, info.num_cores)
  run = pl.kernel(
      body,
      out_type=jax.ShapeDtypeStruct((L, H // _HB, n_tb, _HB, _BB),
                                    jnp.float32),
      mesh=mesh,
      scratch_types=[
          pltpu.VMEM((tasks_per_worker * _BB,), jnp.int32),
          pltpu.VMEM((L, H), jnp.float32),
          pltpu.VMEM((_BB, H), jnp.float32),
          pltpu.VMEM((_BB, H), jnp.float32),
          pltpu.VMEM((H, _OW), jnp.float32),
          pltpu.VMEM((H, _OW), jnp.float32),
          pltpu.SemaphoreType.DMA,
          pltpu.SemaphoreType.DMA,
          pltpu.SemaphoreType.DMA,
          pltpu.SemaphoreType.DMA,
      ],
      compiler_params=pltpu.CompilerParams(use_tc_tiling_on_sc=False,
                                           needs_layout_passes=False),
  )
  xt = jnp.swapaxes(x, 0, 1).reshape(-1)
  out5 = run(xt, emb_table, pos_table)
  return out5.transpose(2, 4, 0, 1, 3).reshape(B, L, H)


# speedup vs baseline: 1.0057x; 1.0057x over previous
import functools

import jax
import jax.numpy as jnp
from jax import lax
from jax.experimental import pallas as pl
from jax.experimental.pallas import tpu as pltpu
from jax.experimental.pallas import tpu_sc as plsc

_BB = 128
_HB = 8
_LANES = 16
_OW = 137


def _body(L, H, B, tasks_per_worker, num_cores,
          xt_hbm, emb_hbm, pos_hbm, out_hbm,
          idx_v, pos_v, rows0, rows1, obuf0, obuf1,
          sem_g0, sem_g1, sem_o0, sem_o1):
  wid = lax.axis_index("s") * num_cores + lax.axis_index("c")
  t0 = wid * tasks_per_worker

  rows = (rows0, rows1)
  obufs = (obuf0, obuf1)
  gsems = (sem_g0, sem_g1)
  osems = (sem_o0, sem_o1)
  n_th = H // _HB

  pltpu.sync_copy(xt_hbm.at[pl.ds(t0 * _BB, tasks_per_worker * _BB)], idx_v)
  pltpu.sync_copy(pos_hbm, pos_v)

  lane = jnp.arange(_LANES, dtype=jnp.int32)
  thv = [(lane + 16 * h4) // _HB for h4 in range(H // _LANES)]
  hhv = [lax.rem(lane + 16 * h4, _HB) for h4 in range(H // _LANES)]

  def gather_into(buf_i, k):
    koff = lax.rem(k, tasks_per_worker)
    return pltpu.async_copy(
        emb_hbm.at[idx_v.at[pl.ds(koff * _BB, _BB)]],
        rows[buf_i], gsems[buf_i])

  def drain_outs(buf_i):
    pltpu.make_async_copy(
        obufs[buf_i].at[:, :, pl.ds(0, _BB)], out_hbm.at[0, :, 0],
        osems[buf_i]).wait()

  def transpose_add(buf_i, l):
    rbuf = rows[buf_i]
    obuf = obufs[buf_i]
    pv = [pos_v[l, pl.ds(h4 * _LANES, _LANES)] for h4 in range(H // _LANES)]

    def four_b(b4, _):
      b0 = b4 * 4
      for i in range(4):
        b = b0 + i
        bs = jnp.full((_LANES,), b, jnp.int32)
        for h4 in range(H // _LANES):
          v = rbuf[b, pl.ds(h4 * _LANES, _LANES)] + pv[h4]
          plsc.store_scatter(obuf, [thv[h4], hhv[h4], bs], v)
      return 0

    lax.fori_loop(0, _BB // 4, four_b, 0)

  def write_out(buf_i, l, tb):
    pltpu.async_copy(
        obufs[buf_i].at[:, :, pl.ds(0, _BB)],
        out_hbm.at[l, :, tb], osems[buf_i])

  n_tb = B // _BB

  gather_into(0, 0)

  def pair(tp, _):
    for k_par in range(2):
      k = tp * 2 + k_par
      t = t0 + k
      l = t // n_tb
      tb = lax.rem(t, n_tb)
      bi = k_par

      pltpu.make_async_copy(
          emb_hbm.at[idx_v.at[pl.ds(0, _BB)]], rows[bi], gsems[bi]).wait()

      @pl.when(k >= 2)
      def _():
        drain_outs(bi)

      gather_into(1 - bi, k + 1)
      transpose_add(bi, l)
      write_out(bi, l, tb)
    return 0

  lax.fori_loop(0, tasks_per_worker // 2, pair, 0)

  pltpu.make_async_copy(
      emb_hbm.at[idx_v.at[pl.ds(0, _BB)]], rows[0], gsems[0]).wait()
  drain_outs(0)
  drain_outs(1)


def kernel(x, emb_table, pos_table):
  B, L = x.shape
  V, H = emb_table.shape
  info = plsc.get_sparse_core_info()
  nw = info.num_cores * info.num_subcores
  n_tb = B // _BB
  tasks_per_worker = (L * n_tb) // nw

  mesh = plsc.VectorSubcoreMesh(core_axis_name="c", subcore_axis_name="s")
  body = functools.partial(_body, L, H, B, tasks_per_worker, info.num_cores)
  run = pl.kernel(
      body,
      out_type=jax.ShapeDtypeStruct((L, H // _HB, n_tb, _HB, _BB),
                                    jnp.float32),
      mesh=mesh,
      scratch_types=[
          pltpu.VMEM((tasks_per_worker * _BB,), jnp.int32),
          pltpu.VMEM((L, H), jnp.float32),
          pltpu.VMEM((_BB, H), jnp.float32),
          pltpu.VMEM((_BB, H), jnp.float32),
          pltpu.VMEM((H // _HB, _HB, _OW), jnp.float32),
          pltpu.VMEM((H // _HB, _HB, _OW), jnp.float32),
          pltpu.SemaphoreType.DMA,
          pltpu.SemaphoreType.DMA,
          pltpu.SemaphoreType.DMA,
          pltpu.SemaphoreType.DMA,
      ],
      compiler_params=pltpu.CompilerParams(use_tc_tiling_on_sc=False,
                                           needs_layout_passes=False),
  )
  xt = jnp.swapaxes(x, 0, 1).reshape(-1)
  out5 = run(xt, emb_table, pos_table)
  return out5.transpose(2, 4, 0, 1, 3).reshape(B, L, H)
